# baseline (speedup 1.0000x reference)
import functools

import jax
import jax.numpy as jnp
from jax import lax
from jax.experimental import pallas as pl
from jax.experimental.pallas import tpu as pltpu
from jax.experimental.pallas import tpu_sc as plsc

_D = 64
_NC = 2
_NS = 16
_NW = _NC * _NS
_BB = 128
_DB = _D // 8


@functools.cache
def _gather_call(seq: int, nbb: int):
    n_units = seq * nbb
    u_per_w = n_units // _NW
    n_super = u_per_w // 4
    mesh = plsc.VectorSubcoreMesh(core_axis_name="c", subcore_axis_name="s")

    @functools.partial(
        pl.kernel,
        out_type=jax.ShapeDtypeStruct((seq, _DB, nbb, 8, _BB), jnp.float32),
        mesh=mesh,
        scratch_types=[
            pltpu.VMEM((u_per_w, _BB), jnp.int32),
            pltpu.VMEM((_BB, _D), jnp.float32),
            pltpu.VMEM((_BB, _D), jnp.float32),
            pltpu.VMEM((_BB, _D), jnp.float32),
            pltpu.VMEM((_BB, _D), jnp.float32),
            pltpu.VMEM((_DB, 8, _BB), jnp.float32),
            pltpu.VMEM((_DB, 8, _BB), jnp.float32),
            pltpu.SemaphoreType.DMA,
            pltpu.SemaphoreType.DMA,
            pltpu.SemaphoreType.DMA,
            pltpu.SemaphoreType.DMA,
            pltpu.SemaphoreType.DMA,
            pltpu.SemaphoreType.DMA,
        ],
        compiler_params=pltpu.CompilerParams(
            use_tc_tiling_on_sc=False, needs_layout_passes=False),
    )
    def body(idx_hbm, table_hbm, out_hbm, idx_all, rows0, rows1, rows2, rows3,
             t0, t1, gsem0, gsem1, gsem2, gsem3, wsem0, wsem1):
        wid = lax.axis_index("s") * _NC + lax.axis_index("c")
        u0 = wid * u_per_w
        rows = (rows0, rows1, rows2, rows3)
        ts = (t0, t1)
        gsems = (gsem0, gsem1, gsem2, gsem3)
        wsems = (wsem0, wsem1)
        lane = lax.iota(jnp.int32, 16)

        pltpu.sync_copy(idx_hbm.at[pl.ds(u0, u_per_w)], idx_all)

        def fire_gather(j, b):
            pltpu.async_copy(table_hbm.at[idx_all.at[j]], rows[b], gsems[b])

        def wait_gather(b):
            pltpu.make_async_copy(
                table_hbm.at[pl.ds(0, _BB)], rows[b], gsems[b]).wait()

        def transpose2(rb, tb):
            src = rows[rb]
            dst = ts[tb]

            def b0_body(b0, carry):
                rv = lane + b0
                for db in range(_DB):
                    for di in range(8):
                        col = jnp.full((16,), 8 * db + di, jnp.int32)
                        v = plsc.load_gather(src, [rv, col])
                        dst[db, di, pl.ds(b0, 16)] = v
                return carry

            lax.fori_loop(0, _BB // 16, lambda g, c: b0_body(g * 16, c), 0)

        def fire_write(j, b):
            u = u0 + j
            s = u // nbb
            bb = u % nbb
            pltpu.async_copy(ts[b], out_hbm.at[s, :, bb], wsems[b])

        def wait_write(b):
            pltpu.make_async_copy(ts[b], out_hbm.at[0, :, 0], wsems[b]).wait()

        for r in range(4):
            fire_gather(r, r)

        def super_body(sidx, carry):
            for r in range(4):
                j = 4 * sidx + r
                wait_gather(r)
                if r < 2:
                    @pl.when(sidx > 0)
                    def _():
                        wait_write(r % 2)
                else:
                    wait_write(r % 2)
                transpose2(r, r % 2)

                @pl.when(sidx < n_super - 1)
                def _():
                    fire_gather(j + 4, r)
                fire_write(j, r % 2)
            return carry

        lax.fori_loop(0, n_super, super_body, 0)
        wait_write(0)
        wait_write(1)

    return body


def kernel(input_ids, embed_table):
    batch, seq = input_ids.shape
    vocab = embed_table.shape[0]
    nbb = batch // _BB
    idx_t = input_ids.T.astype(jnp.int32).reshape(seq * nbb, _BB)
    tab_lin = jax.lax.optimization_barrier(embed_table.reshape(vocab // 2, 2 * _D))
    tab2 = tab_lin.reshape(vocab, _D)
    out5 = _gather_call(seq, nbb)(idx_t, tab2)
    return out5.transpose(2, 4, 0, 1, 3).reshape(batch, seq, _D)


# speedup vs baseline: 1.4798x; 1.4798x over previous
import functools

import jax
import jax.numpy as jnp
from jax import lax
from jax.experimental import pallas as pl
from jax.experimental.pallas import tpu as pltpu
from jax.experimental.pallas import tpu_sc as plsc

_D = 64
_NC = 2
_NS = 16
_NW = _NC * _NS
_KSUB = 128
_NSTREAM = 4
_CHUNK = _KSUB * _NSTREAM


@functools.cache
def _gather_call(n_rows: int):
    b_per_w = n_rows // _NW
    grp_per_w = b_per_w // _KSUB
    n_chunks = b_per_w // _CHUNK
    n_super = n_chunks // 2
    mesh = plsc.VectorSubcoreMesh(core_axis_name="c", subcore_axis_name="s")

    @functools.partial(
        pl.kernel,
        out_type=jax.ShapeDtypeStruct((n_rows, _D), jnp.float32),
        mesh=mesh,
        scratch_types=[
            pltpu.VMEM((grp_per_w, _KSUB), jnp.int32),
            pltpu.VMEM((_CHUNK, _D), jnp.float32),
            pltpu.VMEM((_CHUNK, _D), jnp.float32),
            pltpu.SemaphoreType.DMA,
            pltpu.SemaphoreType.DMA,
            pltpu.SemaphoreType.DMA,
            pltpu.SemaphoreType.DMA,
        ],
        compiler_params=pltpu.CompilerParams(use_tc_tiling_on_sc=False),
    )
    def body(idx_hbm, table_hbm, out_hbm, idx_all, rows0, rows1,
             gsem0, gsem1, wsem0, wsem1):
        wid = lax.axis_index("s") * _NC + lax.axis_index("c")
        grp0 = pl.multiple_of(wid * grp_per_w, grp_per_w)
        out0 = pl.multiple_of(wid * b_per_w, b_per_w)
        rows = (rows0, rows1)
        gsems = (gsem0, gsem1)
        wsems = (wsem0, wsem1)

        pltpu.sync_copy(idx_hbm.at[pl.ds(grp0, grp_per_w)], idx_all)

        def fire_gather(j, b):
            for t in range(_NSTREAM):
                pltpu.async_copy(
                    table_hbm.at[idx_all.at[j * _NSTREAM + t]],
                    rows[b].at[pl.ds(t * _KSUB, _KSUB)],
                    gsems[b],
                )

        def wait_gather(b):
            pltpu.make_async_copy(
                table_hbm.at[pl.ds(0, _CHUNK)], rows[b], gsems[b]).wait()

        def fire_write(j, b):
            off = pl.multiple_of(out0 + j * _CHUNK, _CHUNK)
            pltpu.async_copy(rows[b], out_hbm.at[pl.ds(off, _CHUNK)], wsems[b])

        def wait_write(b):
            pltpu.make_async_copy(
                rows[b], out_hbm.at[pl.ds(0, _CHUNK)], wsems[b]).wait()

        fire_gather(0, 0)

        def super_body(s, carry):
            j = 2 * s

            @pl.when(s > 0)
            def _():
                wait_write(1)
            fire_gather(j + 1, 1)
            wait_gather(0)
            fire_write(j, 0)

            wait_write(0)

            @pl.when(s < n_super - 1)
            def _():
                fire_gather(j + 2, 0)
            wait_gather(1)
            fire_write(j + 1, 1)
            return carry

        lax.fori_loop(0, n_super, super_body, 0)
        wait_write(1)

    return body


def kernel(input_ids, embed_table):
    batch, seq = input_ids.shape
    ids = input_ids.reshape(-1).astype(jnp.int32)
    n_rows = ids.shape[0]
    idx2 = ids.reshape(n_rows // _KSUB, _KSUB)
    tab_rm = jax.lax.optimization_barrier(embed_table.T).T
    out = _gather_call(n_rows)(idx2, tab_rm)
    out_w = jax.lax.optimization_barrier(out.reshape(n_rows // 2, 2 * _D))
    return out_w.reshape(batch, seq, _D)
